# baseline (speedup 1.0000x reference)
import functools

import jax
import jax.numpy as jnp
from jax import lax
from jax.experimental import pallas as pl
from jax.experimental.pallas import tpu as pltpu
from jax.experimental.pallas import tpu_sc as plsc

B = 16384
D = 64
OUTW = 7 * D
BINS = 10000
NF = 6
NC = 2
NS = 16
NW = NC * NS
RPW = B // NW
RT = RPW // 128
NTC = OUTW // 8
NTE = D // 8
L = 16
TILE = 8 * 128
BAND = RT * TILE


def _emb_body(eid_hbm, table_hbm, out_hbm,
              idx_v, rows_v, pair0_v, pair1_v, t16_v,
              gsem, psem0, psem1):
    wid = lax.axis_index("s") * NC + lax.axis_index("c")
    base = wid * RPW
    j0 = wid * RT

    pltpu.sync_copy(eid_hbm.at[pl.ds(base, RPW)], idx_v)

    def _mod_body(i, _):
        v = idx_v[pl.ds(i * L, L)]
        idx_v[pl.ds(i * L, L)] = lax.rem(v, BINS)
        return 0

    lax.fori_loop(0, RPW // L, _mod_body, 0)

    pltpu.async_copy(table_hbm.at[idx_v], rows_v, gsem).wait()

    psems = (psem0, psem1)
    pairs = (pair0_v, pair1_v)
    lanes16 = lax.iota(jnp.int32, L)
    colpat = [(lanes16 + k) % L for k in range(L)]
    qpat = [18 * ((c - lanes16) % L) + lanes16 for c in range(L)]

    def _pair_wait(buf, sem):
        pltpu.make_async_copy(
            pairs[buf], out_hbm.at[pl.ds(0, 2 * BAND)], sem).wait()

    def _emb_pair(p, buf, sem):
        pair = pairs[buf]
        cidx = [cp + p * L for cp in colpat]

        def _rows(q, _):
            rr = q * L
            jj = rr // 128
            sbase = jj * TILE + (rr - jj * 128)
            ridx = lanes16 + rr
            for k in range(L):
                t16_v[pl.ds(k * 18, L)] = plsc.load_gather(
                    rows_v, [ridx, cidx[k]])
            for c in range(L):
                v = plsc.load_gather(t16_v, [qpat[c]])
                pair[pl.ds((c // 8) * BAND + (c % 8) * 128 + sbase, L)] = v
            return 0

        lax.fori_loop(0, RPW // L, _rows, 0)
        pltpu.async_copy(
            pair.at[pl.ds(0, BAND)],
            out_hbm.at[pl.ds((2 * p * 128 + j0) * TILE, BAND)], sem)
        pltpu.async_copy(
            pair.at[pl.ds(BAND, BAND)],
            out_hbm.at[pl.ds(((2 * p + 1) * 128 + j0) * TILE, BAND)], sem)

    for p in range(4):
        if p >= 2:
            _pair_wait(p % 2, psems[p % 2])
        _emb_pair(p, p % 2, psems[p % 2])

    _pair_wait(0, psems[0])
    _pair_wait(1, psems[1])


@functools.partial(
    pl.kernel,
    mesh=plsc.VectorSubcoreMesh(core_axis_name="c", subcore_axis_name="s"),
    out_type=jax.ShapeDtypeStruct((B * D,), jnp.float32),
    compiler_params=pltpu.CompilerParams(use_tc_tiling_on_sc=False,
                                         needs_layout_passes=False),
    scratch_types=[
        pltpu.VMEM((RPW,), jnp.int32),
        pltpu.VMEM((RPW, D), jnp.float32),
        pltpu.VMEM((2 * BAND,), jnp.float32),
        pltpu.VMEM((2 * BAND,), jnp.float32),
        pltpu.VMEM((16 * 18,), jnp.float32),
        pltpu.SemaphoreType.DMA,
        pltpu.SemaphoreType.DMA,
        pltpu.SemaphoreType.DMA,
    ],
)
def _emb_kernel(eid, table, out, idx_v, rows_v, pair0_v, pair1_v, t16_v,
                gsem, psem0, psem1):
    _emb_body(eid, table, out, idx_v, rows_v, pair0_v, pair1_v, t16_v,
              gsem, psem0, psem1)


def _dense_tc_body(feat_ref, w_ref, b_ref, out_ref):
    out_ref[0] = (feat_ref[0][:, None, :] * w_ref[0][None]
                  + b_ref[0][None])


_dense_tc = pl.pallas_call(
    _dense_tc_body,
    grid=(NTC - NTE,),
    in_specs=[
        pl.BlockSpec((1, 128, 128), lambda i: (i // (D // 8), 0, 0)),
        pl.BlockSpec((1, 8, 128), lambda i: (i, 0, 0)),
        pl.BlockSpec((1, 8, 128), lambda i: (i, 0, 0)),
    ],
    out_specs=pl.BlockSpec((1, 128, 8, 128), lambda i: (i + NTE, 0, 0, 0)),
    out_shape=jax.ShapeDtypeStruct((NTC, 128, 8, 128), jnp.float32),
)


def _emb_merge_body(emb_ref, dense_ref, out_ref):
    out_ref[...] = emb_ref[...]


_emb_merge = pl.pallas_call(
    _emb_merge_body,
    grid=(NTE,),
    in_specs=[
        pl.BlockSpec((1, 128, 8, 128), lambda i: (i, 0, 0, 0)),
        pl.BlockSpec((1, 128, 8, 128), lambda i: (i, 0, 0, 0)),
    ],
    out_specs=pl.BlockSpec((1, 128, 8, 128), lambda i: (i, 0, 0, 0)),
    out_shape=jax.ShapeDtypeStruct((NTC, 128, 8, 128), jnp.float32),
    input_output_aliases={1: 0},
)


def kernel(engagement_id, table,
           feat_type, W_type, b_type,
           feat_duration, W_duration, b_duration,
           feat_difficulty, W_difficulty, b_difficulty,
           feat_prerequisites, W_prerequisites, b_prerequisites,
           feat_popularity, W_popularity, b_popularity,
           feat_success_rate, W_success_rate, b_success_rate):
    feats3 = jnp.stack([feat_type, feat_duration, feat_difficulty,
                        feat_prerequisites, feat_popularity,
                        feat_success_rate]).reshape(NF, 128, 128)
    w = jnp.concatenate([W_type[0], W_duration[0], W_difficulty[0],
                         W_prerequisites[0], W_popularity[0],
                         W_success_rate[0]])
    bb = jnp.concatenate([b_type, b_duration, b_difficulty,
                          b_prerequisites, b_popularity, b_success_rate])
    w_bands = jnp.broadcast_to(w.reshape(NTC - NTE, 8)[:, :, None],
                               (NTC - NTE, 8, 128))
    b_bands = jnp.broadcast_to(bb.reshape(NTC - NTE, 8)[:, :, None],
                               (NTC - NTE, 8, 128))

    emb_flat = _emb_kernel(engagement_id, table)
    emb4 = emb_flat.reshape(NTE, 128, 8, 128)
    dense4 = _dense_tc(feats3, w_bands, b_bands)
    out4 = _emb_merge(emb4, dense4)
    return (out4.reshape(NTC, B // 128, 8, 128)
            .transpose(1, 3, 0, 2)
            .reshape(B, OUTW))
